# baseline (speedup 1.0000x reference)
import functools

import jax
import jax.numpy as jnp
from jax import lax
from jax.experimental import pallas as pl
from jax.experimental.pallas import tpu as pltpu
from jax.experimental.pallas import tpu_sc as plsc

B, C, H, W = 2, 128, 192, 256
WIN = 5
STRIDE = 4
GW_SHIFT, GW_MASK = 6, 63
P = WIN * WIN
D = P * C
PR = 32
M = 5000
WP = W + 2
HP = 200
ZROW = H
TROWS = B * HP * WP

NCORES, NSUB = 2, 16
NW = NCORES * NSUB
NB = 8
NBLKS = M // NB
NBIG = NBLKS - 19 * NW
BROWS = NB * P
SPLIT = 104
IDS_LEN = 160
MPAD = 5008


@functools.cache
def _build_sc_gather():
    mesh = plsc.VectorSubcoreMesh(core_axis_name="c", subcore_axis_name="s")
    return functools.partial(
        pl.kernel,
        out_type=jax.ShapeDtypeStruct((M, PR, C), jnp.float32),
        mesh=mesh,
        compiler_params=pltpu.CompilerParams(needs_layout_passes=False),
        scratch_types=[
            pltpu.VMEM((IDS_LEN,), jnp.int32),
            pltpu.VMEM((IDS_LEN,), jnp.int32),
            pltpu.VMEM((20 * BROWS,), jnp.int32),
            pltpu.VMEM((BROWS, C), jnp.float32),
            pltpu.VMEM((BROWS, C), jnp.float32),
            pltpu.VMEM((NB, PR, C), jnp.float32),
            pltpu.VMEM((NB, PR, C), jnp.float32),
            pltpu.SemaphoreType.DMA,
            pltpu.SemaphoreType.DMA,
            pltpu.SemaphoreType.DMA,
            pltpu.SemaphoreType.DMA,
        ],
    )(_sc_gather_body)


def _sc_gather_body(table, b_hbm, ids_hbm, out,
                    b_v, i_v, idx_v, nat_a, nat_b, out_a, out_b,
                    sem_a, sem_b, sem_wa, sem_wb):
    wid = lax.axis_index("s") * NCORES + lax.axis_index("c")
    npair = jnp.where(wid < NBIG, 10, 9)
    m0 = pl.multiple_of(8 * (19 * wid + jnp.minimum(wid, NBIG)), 8)

    pltpu.sync_copy(b_hbm.at[pl.ds(m0, IDS_LEN)], b_v)
    pltpu.sync_copy(ids_hbm.at[pl.ds(m0, IDS_LEN)], i_v)

    lanes = lax.iota(jnp.int32, 16)
    col0 = [(lanes + cg * 16) * P for cg in range(C // 16)]
    mvec = [jnp.full((16,), m, jnp.int32) for m in range(NB)]

    def permute(nat, outb, nm):
        @pl.loop(0, P)
        def _(p):
            for cg in range(C // 16):
                colv = col0[cg] + p
                d1 = lax.shift_right_logical(colv, 7)
                d2 = colv & (C - 1)
                for m in range(nm):
                    v = nat[m * P + p, pl.ds(cg * 16, 16)]
                    plsc.store_scatter(outb, [mvec[m], d1, d2], v)

    def gather_block(table, blk, nat, sem):
        q0 = pl.multiple_of(blk * BROWS, 8)
        c1 = pltpu.async_copy(table.at[idx_v.at[pl.ds(q0, SPLIT)]],
                              nat.at[pl.ds(0, SPLIT)], sem)
        c2 = pltpu.async_copy(table.at[idx_v.at[pl.ds(q0 + SPLIT,
                                                      BROWS - SPLIT)]],
                              nat.at[pl.ds(SPLIT, BROWS - SPLIT)], sem)
        return c1, c2

    def run_feat(ids_v, table, out):
        @pl.loop(0, 10)
        def _(g):
            krel = g * 16 + lanes
            bb = plsc.load_gather(b_v, [krel])
            ii = plsc.load_gather(ids_v, [krel])
            gi = lax.shift_right_logical(ii, GW_SHIFT)
            gj = ii & GW_MASK
            hbase = gi * STRIDE - 2
            wcol = bb * (HP * WP) + gj * STRIDE
            q0 = krel * P
            for p in range(P):
                h = hbase + p // WIN
                if p // WIN < 2:
                    h = jnp.where(h < 0, ZROW, h)
                row = h * WP + wcol + (p % WIN)
                plsc.store_scatter(idx_v, [q0 + p], row)

        @pl.loop(0, npair)
        def _(bp):
            b0 = bp * 2
            a1, a2 = gather_block(table, b0, nat_a, sem_a)
            b1, b2 = gather_block(table, b0 + 1, nat_b, sem_b)
            a1.wait()
            a2.wait()
            permute(nat_a, out_a, NB)
            wa = pltpu.async_copy(out_a, out.at[pl.ds(m0 + b0 * NB, NB)],
                                  sem_wa)
            b1.wait()
            b2.wait()
            permute(nat_b, out_b, NB)
            wb = pltpu.async_copy(out_b,
                                  out.at[pl.ds(m0 + (b0 + 1) * NB, NB)],
                                  sem_wb)
            wa.wait()
            wb.wait()

        @pl.when(wid >= NBIG)
        def _():
            lb = 18
            a1, a2 = gather_block(table, lb, nat_a, sem_a)
            a1.wait()
            a2.wait()
            permute(nat_a, out_a, NB)
            pltpu.sync_copy(out_a, out.at[pl.ds(m0 + lb * NB, NB)])

    run_feat(i_v, table, out)


HB = 8


def _prep_body(x_ref, o_ref):
    hb = pl.program_id(1)

    @pl.when(hb < H // HB)
    def _():
        x = x_ref[0]
        t = x.reshape(C, HB * W).T.reshape(HB, W, C)
        z2 = jnp.zeros((2, C), jnp.float32)
        rows = []
        for hh in range(HB):
            rows.append(z2)
            rows.append(t[hh])
        o_ref[...] = jnp.concatenate(rows, axis=0)

    @pl.when(hb >= H // HB)
    def _():
        o_ref[...] = jnp.zeros((HB * WP, C), jnp.float32)


@functools.cache
def _build_prep():
    return pl.pallas_call(
        _prep_body,
        grid=(B, HP // HB),
        in_specs=[pl.BlockSpec(
            (1, C, HB, W),
            lambda b, hb: (b, 0, jnp.minimum(hb, H // HB - 1), 0))],
        out_specs=pl.BlockSpec((HB * WP, C),
                               lambda b, hb: (b * (HP // HB) + hb, 0)),
        out_shape=jax.ShapeDtypeStruct((TROWS, C), jnp.float32),
    )


def kernel(feat_f0, feat_f1, hw0_f, hw0_c, b_ids, i_ids, j_ids):
    prep = _build_prep()
    sc = _build_sc_gather()
    pad = (0, MPAD - M)
    b = jnp.pad(b_ids.astype(jnp.int32), pad)
    i = jnp.pad(i_ids.astype(jnp.int32), pad)
    j = jnp.pad(j_ids.astype(jnp.int32), pad)
    t0 = prep(feat_f0)
    g0 = sc(t0, b, i)
    t1 = prep(feat_f1)
    g1 = sc(t1, b, j)
    return (g0[:, :P, :], g1[:, :P, :])


# speedup vs baseline: 1.0394x; 1.0394x over previous
import functools

import jax
import jax.numpy as jnp
from jax import lax
from jax.experimental import pallas as pl
from jax.experimental.pallas import tpu as pltpu
from jax.experimental.pallas import tpu_sc as plsc

B, C, H, W = 2, 128, 192, 256
WIN = 5
STRIDE = 4
GW_SHIFT, GW_MASK = 6, 63
P = WIN * WIN
D = P * C
PR = 32
M = 5000
WP = W + 2
HP = 200
ZROW = H
TROWS = B * HP * WP

NCORES, NSUB = 2, 16
NW = NCORES * NSUB
NB = 8
NBLKS = M // NB
NBIG = NBLKS - 19 * NW
BROWS = NB * P
SPLIT = 104
IDS_LEN = 160
MPAD = 5008


@functools.cache
def _build_sc_gather():
    mesh = plsc.VectorSubcoreMesh(core_axis_name="c", subcore_axis_name="s")
    return functools.partial(
        pl.kernel,
        out_type=jax.ShapeDtypeStruct((M, PR, C), jnp.float32),
        mesh=mesh,
        compiler_params=pltpu.CompilerParams(needs_layout_passes=False),
        scratch_types=[
            pltpu.VMEM((IDS_LEN,), jnp.int32),
            pltpu.VMEM((IDS_LEN,), jnp.int32),
            pltpu.VMEM((20 * BROWS,), jnp.int32),
            pltpu.VMEM((BROWS, C), jnp.float32),
            pltpu.VMEM((BROWS, C), jnp.float32),
            pltpu.VMEM((NB, PR, C), jnp.float32),
            pltpu.VMEM((NB, PR, C), jnp.float32),
            pltpu.SemaphoreType.DMA,
            pltpu.SemaphoreType.DMA,
            pltpu.SemaphoreType.DMA,
            pltpu.SemaphoreType.DMA,
        ],
    )(_sc_gather_body)


def _sc_gather_body(table, b_hbm, ids_hbm, out,
                    b_v, i_v, idx_v, nat_a, nat_b, out_a, out_b,
                    sem_a, sem_b, sem_wa, sem_wb):
    wid = lax.axis_index("s") * NCORES + lax.axis_index("c")
    npair = jnp.where(wid < NBIG, 10, 9)
    m0 = pl.multiple_of(8 * (19 * wid + jnp.minimum(wid, NBIG)), 8)

    pltpu.sync_copy(b_hbm.at[pl.ds(m0, IDS_LEN)], b_v)
    pltpu.sync_copy(ids_hbm.at[pl.ds(m0, IDS_LEN)], i_v)

    lanes = lax.iota(jnp.int32, 16)
    col0 = [(lanes + cg * 16) * P for cg in range(C // 16)]
    mvec = [jnp.full((16,), m, jnp.int32) for m in range(NB)]

    def permute(nat, outb, nm):
        @pl.loop(0, P)
        def _(p):
            for cg in range(C // 16):
                colv = col0[cg] + p
                d1 = lax.shift_right_logical(colv, 7)
                d2 = colv & (C - 1)
                for m in range(nm):
                    v = nat[m * P + p, pl.ds(cg * 16, 16)]
                    plsc.store_scatter(outb, [mvec[m], d1, d2], v)

    def gather_block(table, blk, nat, sem):
        q0 = pl.multiple_of(blk * BROWS, 8)
        c1 = pltpu.async_copy(table.at[idx_v.at[pl.ds(q0, SPLIT)]],
                              nat.at[pl.ds(0, SPLIT)], sem)
        c2 = pltpu.async_copy(table.at[idx_v.at[pl.ds(q0 + SPLIT,
                                                      BROWS - SPLIT)]],
                              nat.at[pl.ds(SPLIT, BROWS - SPLIT)], sem)
        return c1, c2

    def run_feat(ids_v, table, out):
        @pl.loop(0, 10)
        def _(g):
            krel = g * 16 + lanes
            bb = plsc.load_gather(b_v, [krel])
            ii = plsc.load_gather(ids_v, [krel])
            gi = lax.shift_right_logical(ii, GW_SHIFT)
            gj = ii & GW_MASK
            hbase = gi * STRIDE - 2
            wcol = bb * (HP * WP) + gj * STRIDE
            q0 = krel * P
            for p in range(P):
                h = hbase + p // WIN
                if p // WIN < 2:
                    h = jnp.where(h < 0, ZROW, h)
                row = h * WP + wcol + (p % WIN)
                plsc.store_scatter(idx_v, [q0 + p], row)

        @pl.loop(0, npair)
        def _(bp):
            b0 = bp * 2
            a1, a2 = gather_block(table, b0, nat_a, sem_a)
            b1, b2 = gather_block(table, b0 + 1, nat_b, sem_b)
            a1.wait()
            a2.wait()
            permute(nat_a, out_a, NB)
            wa = pltpu.async_copy(out_a, out.at[pl.ds(m0 + b0 * NB, NB)],
                                  sem_wa)
            b1.wait()
            b2.wait()
            permute(nat_b, out_b, NB)
            wb = pltpu.async_copy(out_b,
                                  out.at[pl.ds(m0 + (b0 + 1) * NB, NB)],
                                  sem_wb)
            wa.wait()
            wb.wait()

        @pl.when(wid >= NBIG)
        def _():
            lb = 18
            a1, a2 = gather_block(table, lb, nat_a, sem_a)
            a1.wait()
            a2.wait()
            permute(nat_a, out_a, NB)
            pltpu.sync_copy(out_a, out.at[pl.ds(m0 + lb * NB, NB)])

    run_feat(i_v, table, out)


HB = 8


def _prep_body(x_ref, o_ref):
    hb = pl.program_id(1)

    @pl.when(hb < H // HB)
    def _():
        x = x_ref[0]
        t = x.reshape(C, HB * W).T.reshape(HB, W, C)
        z2 = jnp.zeros((2, C), jnp.float32)
        rows = []
        for hh in range(HB):
            rows.append(z2)
            rows.append(t[hh])
        o_ref[...] = jnp.concatenate(rows, axis=0)

    @pl.when(hb >= H // HB)
    def _():
        o_ref[...] = jnp.zeros((HB * WP, C), jnp.float32)


@functools.cache
def _build_prep():
    return pl.pallas_call(
        _prep_body,
        grid=(B, HP // HB),
        in_specs=[pl.BlockSpec(
            (1, C, HB, W),
            lambda b, hb: (b, 0, jnp.minimum(hb, H // HB - 1), 0))],
        out_specs=pl.BlockSpec((HB * WP, C),
                               lambda b, hb: (b * (HP // HB) + hb, 0)),
        out_shape=jax.ShapeDtypeStruct((TROWS, C), jnp.float32),
    )


EB = 40


def _slice_body(x_ref, o_ref):
    o_ref[...] = x_ref[:, :P, :]


@functools.cache
def _build_slice():
    return pl.pallas_call(
        _slice_body,
        grid=(M // EB,),
        in_specs=[pl.BlockSpec((EB, PR, C), lambda m: (m, 0, 0))],
        out_specs=pl.BlockSpec((EB, P, C), lambda m: (m, 0, 0)),
        out_shape=jax.ShapeDtypeStruct((M, P, C), jnp.float32),
    )


def kernel(feat_f0, feat_f1, hw0_f, hw0_c, b_ids, i_ids, j_ids):
    prep = _build_prep()
    sc = _build_sc_gather()
    sl = _build_slice()
    pad = (0, MPAD - M)
    b = jnp.pad(b_ids.astype(jnp.int32), pad)
    i = jnp.pad(i_ids.astype(jnp.int32), pad)
    j = jnp.pad(j_ids.astype(jnp.int32), pad)
    t0 = prep(feat_f0)
    g0 = sc(t0, b, i)
    t1 = prep(feat_f1)
    g1 = sc(t1, b, j)
    return (sl(g0), sl(g1))


# speedup vs baseline: 1.0560x; 1.0159x over previous
import functools

import jax
import jax.numpy as jnp
from jax import lax
from jax.experimental import pallas as pl
from jax.experimental.pallas import tpu as pltpu
from jax.experimental.pallas import tpu_sc as plsc

B, C, H, W = 2, 128, 192, 256
WIN = 5
STRIDE = 4
GW_SHIFT, GW_MASK = 6, 63
P = WIN * WIN
D = P * C
M = 5000
WP = W + 2
HP = 200
ZROW = H
TROWS = B * HP * WP

NCORES, NSUB = 2, 16
NW = NCORES * NSUB
NB = 8
NBLKS = M // NB
NBIG = NBLKS - 19 * NW
BROWS = NB * P
SPLIT = 104
IDS_LEN = 160
MPAD = 5008


@functools.cache
def _build_sc_gather():
    mesh = plsc.VectorSubcoreMesh(core_axis_name="c", subcore_axis_name="s")
    return functools.partial(
        pl.kernel,
        out_type=jax.ShapeDtypeStruct((M, D), jnp.float32),
        mesh=mesh,
        compiler_params=pltpu.CompilerParams(needs_layout_passes=False),
        scratch_types=[
            pltpu.VMEM((IDS_LEN,), jnp.int32),
            pltpu.VMEM((IDS_LEN,), jnp.int32),
            pltpu.VMEM((20 * BROWS,), jnp.int32),
            pltpu.VMEM((BROWS, C), jnp.float32),
            pltpu.VMEM((BROWS, C), jnp.float32),
            pltpu.VMEM((NB, D), jnp.float32),
            pltpu.VMEM((NB, D), jnp.float32),
            pltpu.SemaphoreType.DMA,
            pltpu.SemaphoreType.DMA,
            pltpu.SemaphoreType.DMA,
            pltpu.SemaphoreType.DMA,
        ],
    )(_sc_gather_body)


def _sc_gather_body(table, b_hbm, ids_hbm, out,
                    b_v, i_v, idx_v, nat_a, nat_b, out_a, out_b,
                    sem_a, sem_b, sem_wa, sem_wb):
    wid = lax.axis_index("s") * NCORES + lax.axis_index("c")
    npair = jnp.where(wid < NBIG, 10, 9)
    m0 = pl.multiple_of(8 * (19 * wid + jnp.minimum(wid, NBIG)), 8)

    pltpu.sync_copy(b_hbm.at[pl.ds(m0, IDS_LEN)], b_v)
    pltpu.sync_copy(ids_hbm.at[pl.ds(m0, IDS_LEN)], i_v)

    lanes = lax.iota(jnp.int32, 16)
    col0 = [(lanes + cg * 16) * P for cg in range(C // 16)]
    mvec = [jnp.full((16,), m, jnp.int32) for m in range(NB)]

    def permute(nat, outb, nm):
        @pl.loop(0, P)
        def _(p):
            for cg in range(C // 16):
                colv = col0[cg] + p
                for m in range(nm):
                    v = nat[m * P + p, pl.ds(cg * 16, 16)]
                    plsc.store_scatter(outb, [mvec[m], colv], v)

    def gather_block(table, blk, nat, sem):
        q0 = pl.multiple_of(blk * BROWS, 8)
        c1 = pltpu.async_copy(table.at[idx_v.at[pl.ds(q0, SPLIT)]],
                              nat.at[pl.ds(0, SPLIT)], sem)
        c2 = pltpu.async_copy(table.at[idx_v.at[pl.ds(q0 + SPLIT,
                                                      BROWS - SPLIT)]],
                              nat.at[pl.ds(SPLIT, BROWS - SPLIT)], sem)
        return c1, c2

    def run_feat(ids_v, table, out):
        @pl.loop(0, 10)
        def _(g):
            krel = g * 16 + lanes
            bb = plsc.load_gather(b_v, [krel])
            ii = plsc.load_gather(ids_v, [krel])
            gi = lax.shift_right_logical(ii, GW_SHIFT)
            gj = ii & GW_MASK
            hbase = gi * STRIDE - 2
            wcol = bb * (HP * WP) + gj * STRIDE
            q0 = krel * P
            for p in range(P):
                h = hbase + p // WIN
                if p // WIN < 2:
                    h = jnp.where(h < 0, ZROW, h)
                row = h * WP + wcol + (p % WIN)
                plsc.store_scatter(idx_v, [q0 + p], row)

        @pl.loop(0, npair)
        def _(bp):
            b0 = bp * 2
            a1, a2 = gather_block(table, b0, nat_a, sem_a)
            b1, b2 = gather_block(table, b0 + 1, nat_b, sem_b)
            a1.wait()
            a2.wait()
            permute(nat_a, out_a, NB)
            wa = pltpu.async_copy(out_a, out.at[pl.ds(m0 + b0 * NB, NB)],
                                  sem_wa)
            b1.wait()
            b2.wait()
            permute(nat_b, out_b, NB)
            wb = pltpu.async_copy(out_b,
                                  out.at[pl.ds(m0 + (b0 + 1) * NB, NB)],
                                  sem_wb)
            wa.wait()
            wb.wait()

        @pl.when(wid >= NBIG)
        def _():
            lb = 18
            a1, a2 = gather_block(table, lb, nat_a, sem_a)
            a1.wait()
            a2.wait()
            permute(nat_a, out_a, NB)
            pltpu.sync_copy(out_a, out.at[pl.ds(m0 + lb * NB, NB)])

    run_feat(i_v, table, out)


HB = 8


def _prep_body(x_ref, o_ref):
    hb = pl.program_id(1)

    @pl.when(hb < H // HB)
    def _():
        x = x_ref[0]
        t = x.reshape(C, HB * W).T.reshape(HB, W, C)
        z2 = jnp.zeros((2, C), jnp.float32)
        rows = []
        for hh in range(HB):
            rows.append(z2)
            rows.append(t[hh])
        o_ref[...] = jnp.concatenate(rows, axis=0)

    @pl.when(hb >= H // HB)
    def _():
        o_ref[...] = jnp.zeros((HB * WP, C), jnp.float32)


@functools.cache
def _build_prep():
    return pl.pallas_call(
        _prep_body,
        grid=(B, HP // HB),
        in_specs=[pl.BlockSpec(
            (1, C, HB, W),
            lambda b, hb: (b, 0, jnp.minimum(hb, H // HB - 1), 0))],
        out_specs=pl.BlockSpec((HB * WP, C),
                               lambda b, hb: (b * (HP // HB) + hb, 0)),
        out_shape=jax.ShapeDtypeStruct((TROWS, C), jnp.float32),
    )


EB = 40


def _epi_body(x_ref, o_ref):
    o_ref[...] = x_ref[...].reshape(EB, P, C)


@functools.cache
def _build_epi():
    return pl.pallas_call(
        _epi_body,
        grid=(M // EB,),
        in_specs=[pl.BlockSpec((EB, D), lambda m: (m, 0))],
        out_specs=pl.BlockSpec((EB, P, C), lambda m: (m, 0, 0)),
        out_shape=jax.ShapeDtypeStruct((M, P, C), jnp.float32),
    )


def kernel(feat_f0, feat_f1, hw0_f, hw0_c, b_ids, i_ids, j_ids):
    prep = _build_prep()
    sc = _build_sc_gather()
    epi = _build_epi()
    pad = (0, MPAD - M)
    b = jnp.pad(b_ids.astype(jnp.int32), pad)
    i = jnp.pad(i_ids.astype(jnp.int32), pad)
    j = jnp.pad(j_ids.astype(jnp.int32), pad)
    t0 = prep(feat_f0)
    g0 = sc(t0, b, i)
    t1 = prep(feat_f1)
    g1 = sc(t1, b, j)
    return (epi(g0), epi(g1))
